# Initial kernel scaffold; baseline (speedup 1.0000x reference)
import jax
import jax.numpy as jnp
from jax.experimental import pallas as pl


def kernel(feats, table):
    raise NotImplementedError("write your pallas kernel here")


# speedup vs baseline: 2.9131x; 2.9131x over previous
import jax
import jax.numpy as jnp
from jax import lax
from jax.experimental import pallas as pl
from jax.experimental.pallas import tpu as pltpu
from jax.experimental.pallas import tpu_sc as plsc

DIM = 512
NW = 32
CHUNK = 96


def _gather_body(feats_hbm, table_hbm, out_hbm, idx_v, buf, sem):
    wid = lax.axis_index("s") * 2 + lax.axis_index("c")
    n = feats_hbm.shape[0]
    per_w = n // NW
    chunks = per_w // CHUNK

    def step(i, carry):
        base = wid * per_w + i * CHUNK
        pltpu.sync_copy(feats_hbm.at[pl.ds(base, CHUNK)], idx_v)
        pltpu.async_copy(table_hbm.at[idx_v], buf, sem).wait()
        pltpu.sync_copy(buf, out_hbm.at[pl.ds(base, CHUNK)])
        return carry

    lax.fori_loop(0, chunks, step, 0)


def kernel(feats, table):
    B, T = feats.shape
    flat = feats.reshape(B * T)
    mesh = plsc.VectorSubcoreMesh(core_axis_name="c", subcore_axis_name="s")
    out = pl.kernel(
        _gather_body,
        mesh=mesh,
        out_type=jax.ShapeDtypeStruct((B * T, DIM), jnp.float32),
        scratch_types=[
            pltpu.VMEM((CHUNK,), jnp.int32),
            pltpu.VMEM((CHUNK, DIM), jnp.float32),
            pltpu.SemaphoreType.DMA,
        ],
    )(flat, table)
    return out.reshape(B, T, DIM)


# speedup vs baseline: 3.5059x; 1.2035x over previous
import jax
import jax.numpy as jnp
from jax import lax
from jax.experimental import pallas as pl
from jax.experimental.pallas import tpu as pltpu
from jax.experimental.pallas import tpu_sc as plsc

DIM = 512
NW = 32
CHUNK = 96


def _gather_body(feats_hbm, table_hbm, out_hbm,
                 idx_all, buf0, buf1, gs0, gs1, ss0, ss1):
    wid = lax.axis_index("s") * 2 + lax.axis_index("c")
    n = feats_hbm.shape[0]
    per_w = n // NW
    chunks = per_w // CHUNK
    start = wid * per_w

    bufs = (buf0, buf1)
    gs = (gs0, gs1)
    ss = (ss0, ss1)

    def out_slc(i):
        return out_hbm.at[pl.ds(start + i * CHUNK, CHUNK)]

    def idx_slc(i):
        return idx_all.at[pl.ds(i * CHUNK, CHUNK)]

    def gather(i, b):
        return pltpu.async_copy(table_hbm.at[idx_slc(i)], bufs[b], gs[b])

    def gather_wait(i, b):
        pltpu.make_async_copy(table_hbm.at[idx_slc(i)], bufs[b], gs[b]).wait()

    def store(i, b):
        return pltpu.async_copy(bufs[b], out_slc(i), ss[b])

    def store_wait(i, b):
        pltpu.make_async_copy(bufs[b], out_slc(i), ss[b]).wait()

    pltpu.sync_copy(feats_hbm.at[pl.ds(start, per_w)], idx_all)

    gather(0, 0)
    gather(1, 1)
    gather_wait(0, 0)
    store(0, 0)

    def step(j, carry):
        i = 2 * j + 1
        store_wait(i - 1, 0)
        gather(i + 1, 0)
        gather_wait(i, 1)
        store(i, 1)
        store_wait(i, 1)
        gather(i + 2, 1)
        gather_wait(i + 1, 0)
        store(i + 1, 0)
        return carry

    lax.fori_loop(0, (chunks - 2) // 2, step, 0)

    gather_wait(chunks - 1, 1)
    store(chunks - 1, 1)
    store_wait(chunks - 2, 0)
    store_wait(chunks - 1, 1)


def kernel(feats, table):
    B, T = feats.shape
    flat = feats.reshape(B * T)
    per_w = (B * T) // NW
    mesh = plsc.VectorSubcoreMesh(core_axis_name="c", subcore_axis_name="s")
    out = pl.kernel(
        _gather_body,
        mesh=mesh,
        out_type=jax.ShapeDtypeStruct((B * T, DIM), jnp.float32),
        scratch_types=[
            pltpu.VMEM((per_w,), jnp.int32),
            pltpu.VMEM((CHUNK, DIM), jnp.float32),
            pltpu.VMEM((CHUNK, DIM), jnp.float32),
            pltpu.SemaphoreType.DMA,
            pltpu.SemaphoreType.DMA,
            pltpu.SemaphoreType.DMA,
            pltpu.SemaphoreType.DMA,
        ],
    )(flat, table)
    return out.reshape(B, T, DIM)


# speedup vs baseline: 3.5112x; 1.0015x over previous
import jax
import jax.numpy as jnp
from jax import lax
from jax.experimental import pallas as pl
from jax.experimental.pallas import tpu as pltpu
from jax.experimental.pallas import tpu_sc as plsc

DIM = 512
NW = 32
CHUNK = 64
NBUF = 3


def _gather_body(feats_hbm, table_hbm, out_hbm,
                 idx_all, buf0, buf1, buf2, gs0, gs1, gs2, ss0, ss1, ss2):
    wid = lax.axis_index("s") * 2 + lax.axis_index("c")
    n = feats_hbm.shape[0]
    per_w = n // NW
    chunks = per_w // CHUNK
    start = wid * per_w

    bufs = (buf0, buf1, buf2)
    gs = (gs0, gs1, gs2)
    ss = (ss0, ss1, ss2)

    def out_slc(i):
        return out_hbm.at[pl.ds(start + i * CHUNK, CHUNK)]

    def idx_slc(i):
        return idx_all.at[pl.ds(i * CHUNK, CHUNK)]

    def gather(i, b):
        pltpu.async_copy(table_hbm.at[idx_slc(i)], bufs[b], gs[b])

    def gather_wait(i, b):
        pltpu.make_async_copy(table_hbm.at[idx_slc(i)], bufs[b], gs[b]).wait()

    def store(i, b):
        pltpu.async_copy(bufs[b], out_slc(i), ss[b])

    def store_wait(i, b):
        pltpu.make_async_copy(bufs[b], out_slc(i), ss[b]).wait()

    pltpu.sync_copy(feats_hbm.at[pl.ds(start, per_w)], idx_all)

    gather(0, 0)
    gather(1, 1)

    gather_wait(0, 0)
    store(0, 0)
    gather(2, 2)

    def body(k, b, b2):
        gather_wait(k, b)
        store(k, b)
        store_wait(k - 1, b2)
        gather(k + 2, b2)

    def step(j, carry):
        k = 3 * j + 1
        body(k, 1, 0)
        body(k + 1, 2, 1)
        body(k + 2, 0, 2)
        return carry

    lax.fori_loop(0, (chunks - 3) // 3, step, 0)

    gather_wait(chunks - 2, 1)
    store(chunks - 2, 1)
    store_wait(chunks - 3, 0)
    gather_wait(chunks - 1, 2)
    store(chunks - 1, 2)
    store_wait(chunks - 2, 1)
    store_wait(chunks - 1, 2)


def kernel(feats, table):
    B, T = feats.shape
    flat = feats.reshape(B * T)
    per_w = (B * T) // NW
    mesh = plsc.VectorSubcoreMesh(core_axis_name="c", subcore_axis_name="s")
    out = pl.kernel(
        _gather_body,
        mesh=mesh,
        out_type=jax.ShapeDtypeStruct((B * T, DIM), jnp.float32),
        scratch_types=[
            pltpu.VMEM((per_w,), jnp.int32),
            pltpu.VMEM((CHUNK, DIM), jnp.float32),
            pltpu.VMEM((CHUNK, DIM), jnp.float32),
            pltpu.VMEM((CHUNK, DIM), jnp.float32),
            pltpu.SemaphoreType.DMA,
            pltpu.SemaphoreType.DMA,
            pltpu.SemaphoreType.DMA,
            pltpu.SemaphoreType.DMA,
            pltpu.SemaphoreType.DMA,
            pltpu.SemaphoreType.DMA,
        ],
    )(flat, table)
    return out.reshape(B, T, DIM)


# speedup vs baseline: 3.5164x; 1.0015x over previous
import jax
import jax.numpy as jnp
from jax import lax
from jax.experimental import pallas as pl
from jax.experimental.pallas import tpu as pltpu
from jax.experimental.pallas import tpu_sc as plsc

DIM = 512
NW = 32
CHUNK = 48
NBUF = 5


def _gather_body(feats_hbm, table_hbm, out_hbm, idx_all,
                 buf0, buf1, buf2, buf3, buf4,
                 gs0, gs1, gs2, gs3, gs4,
                 ss0, ss1, ss2, ss3, ss4):
    wid = lax.axis_index("s") * 2 + lax.axis_index("c")
    n = feats_hbm.shape[0]
    per_w = n // NW
    chunks = per_w // CHUNK
    start = wid * per_w

    bufs = (buf0, buf1, buf2, buf3, buf4)
    gs = (gs0, gs1, gs2, gs3, gs4)
    ss = (ss0, ss1, ss2, ss3, ss4)

    def out_slc(i):
        return out_hbm.at[pl.ds(start + i * CHUNK, CHUNK)]

    def idx_slc(i):
        return idx_all.at[pl.ds(i * CHUNK, CHUNK)]

    def gather(i, b):
        pltpu.async_copy(table_hbm.at[idx_slc(i)], bufs[b], gs[b])

    def gather_wait(i, b):
        pltpu.make_async_copy(table_hbm.at[idx_slc(i)], bufs[b], gs[b]).wait()

    def store(i, b):
        pltpu.async_copy(bufs[b], out_slc(i), ss[b])

    def store_wait(i, b):
        pltpu.make_async_copy(bufs[b], out_slc(i), ss[b]).wait()

    pltpu.sync_copy(feats_hbm.at[pl.ds(start, per_w)], idx_all)

    gather(0, 0)
    gather(1, 1)
    gather(2, 2)

    gather_wait(0, 0)
    store(0, 0)
    gather(3, 3)
    gather_wait(1, 1)
    store(1, 1)
    gather(4, 4)
    gather_wait(2, 2)
    store(2, 2)
    store_wait(0, 0)
    gather(5, 0)

    def body(k, b, b2):
        gather_wait(k, b)
        store(k, b)
        store_wait(k - 2, b2)
        gather(k + 3, b2)

    def step(j, carry):
        k = 5 * j + 3
        body(k, 3, 1)
        body(k + 1, 4, 2)
        body(k + 2, 0, 3)
        body(k + 3, 1, 4)
        body(k + 4, 2, 0)
        return carry

    lax.fori_loop(0, (chunks - 6) // 5, step, 0)

    gather_wait(chunks - 3, 3)
    store(chunks - 3, 3)
    store_wait(chunks - 5, 1)
    gather_wait(chunks - 2, 4)
    store(chunks - 2, 4)
    store_wait(chunks - 4, 2)
    gather_wait(chunks - 1, 0)
    store(chunks - 1, 0)
    store_wait(chunks - 3, 3)
    store_wait(chunks - 2, 4)
    store_wait(chunks - 1, 0)


def kernel(feats, table):
    B, T = feats.shape
    flat = feats.reshape(B * T)
    per_w = (B * T) // NW
    mesh = plsc.VectorSubcoreMesh(core_axis_name="c", subcore_axis_name="s")
    out = pl.kernel(
        _gather_body,
        mesh=mesh,
        out_type=jax.ShapeDtypeStruct((B * T, DIM), jnp.float32),
        scratch_types=(
            [pltpu.VMEM((per_w,), jnp.int32)]
            + [pltpu.VMEM((CHUNK, DIM), jnp.float32)] * NBUF
            + [pltpu.SemaphoreType.DMA] * (2 * NBUF)
        ),
    )(flat, table)
    return out.reshape(B, T, DIM)
